# baseline (speedup 1.0000x reference)
import jax
import jax.numpy as jnp
import numpy as np
from jax.experimental import pallas as pl

_B, _L, _C, _NS = 1, 8192, 8, 64

_KS0 = np.uint32(0)
_KS1 = np.uint32(1234)
_KS2 = np.uint32(_KS0 ^ _KS1 ^ np.uint32(0x1BD11BDA))
_TINY = np.float32(np.finfo(np.float32).tiny)
_ROT_A = (13, 15, 26, 6)
_ROT_B = (17, 29, 16, 24)


def _rotl(x, r):
    return (x << np.uint32(r)) | (x >> np.uint32(32 - r))


def _threefry_rounds(x0, x1, rots):
    for r in rots:
        x0 = x0 + x1
        x1 = _rotl(x1, r)
        x1 = x0 ^ x1
    return x0, x1


def _threefry_bits(x1):
    x0 = jnp.full(x1.shape, _KS0, jnp.uint32)
    x0, x1 = _threefry_rounds(x0, x1, _ROT_A)
    x0, x1 = x0 + _KS1, x1 + (_KS2 + np.uint32(1))
    x0, x1 = _threefry_rounds(x0, x1, _ROT_B)
    x0, x1 = x0 + _KS2, x1 + (_KS0 + np.uint32(2))
    x0, x1 = _threefry_rounds(x0, x1, _ROT_A)
    x0, x1 = x0 + _KS0, x1 + (_KS1 + np.uint32(3))
    x0, x1 = _threefry_rounds(x0, x1, _ROT_B)
    x0, x1 = x0 + _KS1, x1 + (_KS2 + np.uint32(4))
    x0, x1 = _threefry_rounds(x0, x1, _ROT_A)
    x0, x1 = x0 + _KS2, x1 + (_KS0 + np.uint32(5))
    return x0 ^ x1


_R = np.arange(_C, dtype=np.uint32)[:, None]
_LN = np.arange(_L, dtype=np.uint32)[None, :]
_P = ((_R + 6) & 7) >> 2
_CA = (_R + 2) & 3
_IDXC = ((_P << 16) | (_LN << 3) | (_CA + 2)) + _KS1
del _R, _LN, _P, _CA

_PAIRS = 8


def _sample_kernel(lg_ref, ic_ref, out_ref):
    i = pl.program_id(0)
    shape = (_C, _L)
    row = jax.lax.broadcasted_iota(jnp.uint32, shape, 0)

    lg = lg_ref[...]
    act = (row >= np.uint32(2)) & (row < np.uint32(6))
    m4 = jnp.where(act, lg, jnp.roll(lg, 4, axis=0))
    ic = ic_ref[...]

    for u in range(_PAIRS):
        base = ((i * _PAIRS + u) * (2 * _L * _C)).astype(jnp.uint32)
        bits = _threefry_bits(ic + base)
        fbits = (bits >> np.uint32(9)) | np.uint32(0x3F800000)
        floats = (jax.lax.bitcast_convert_type(fbits, jnp.float32)
                  - np.float32(1.0))
        u01 = floats + _TINY
        g = -jnp.log(-jnp.log(u01))
        s = g + m4

        m = s
        for k, sel in ((1, (row & np.uint32(1)) == 0),
                       (2, ((row + np.uint32(2)) & np.uint32(2)) == 0)):
            fwd = jnp.roll(m, -k, axis=0)
            bwd = jnp.roll(m, k, axis=0)
            m = jnp.maximum(m, jnp.where(sel, fwd, bwd))

        oh = jnp.where(s == m, np.float32(1.0), np.float32(0.0))
        out_ref[2 * u] = jnp.where(act, oh, np.float32(0.0))
        out_ref[2 * u + 1] = jnp.where(act, jnp.roll(oh, 4, axis=0),
                                       np.float32(0.0))


def kernel(logits):
    lg = jnp.transpose(logits[0])
    out = pl.pallas_call(
        _sample_kernel,
        grid=(_NS // (2 * _PAIRS),),
        in_specs=[pl.BlockSpec((_C, _L), lambda i: (0, 0)),
                  pl.BlockSpec((_C, _L), lambda i: (0, 0))],
        out_specs=pl.BlockSpec((2 * _PAIRS, _C, _L), lambda i: (i, 0, 0)),
        out_shape=jax.ShapeDtypeStruct((_NS, _C, _L), jnp.float32),
    )(lg, jnp.asarray(_IDXC))
    return jnp.transpose(out, (0, 2, 1)).reshape(_B, _NS, _L, _C)


# speedup vs baseline: 1.0242x; 1.0242x over previous
import jax
import jax.numpy as jnp
import numpy as np
from jax.experimental import pallas as pl

_B, _L, _C, _NS = 1, 8192, 8, 64

_KS0 = np.uint32(0)
_KS1 = np.uint32(1234)
_KS2 = np.uint32(_KS0 ^ _KS1 ^ np.uint32(0x1BD11BDA))
_TINY = np.float32(np.finfo(np.float32).tiny)
_ROT_A = (13, 15, 26, 6)
_ROT_B = (17, 29, 16, 24)


def _rotl(x, r):
    return (x << np.uint32(r)) | (x >> np.uint32(32 - r))


def _threefry_rounds(x0, x1, rots):
    for r in rots:
        x0 = x0 + x1
        x1 = _rotl(x1, r)
        x1 = x0 ^ x1
    return x0, x1


def _threefry_bits(x1):
    x0 = jnp.full(x1.shape, _KS0, jnp.uint32)
    x0, x1 = _threefry_rounds(x0, x1, _ROT_A)
    x0, x1 = x0 + _KS1, x1 + (_KS2 + np.uint32(1))
    x0, x1 = _threefry_rounds(x0, x1, _ROT_B)
    x0, x1 = x0 + _KS2, x1 + (_KS0 + np.uint32(2))
    x0, x1 = _threefry_rounds(x0, x1, _ROT_A)
    x0, x1 = x0 + _KS0, x1 + (_KS1 + np.uint32(3))
    x0, x1 = _threefry_rounds(x0, x1, _ROT_B)
    x0, x1 = x0 + _KS1, x1 + (_KS2 + np.uint32(4))
    x0, x1 = _threefry_rounds(x0, x1, _ROT_A)
    x0, x1 = x0 + _KS2, x1 + (_KS0 + np.uint32(5))
    return x0 ^ x1


_R = np.arange(_C, dtype=np.uint32)[:, None]
_LN = np.arange(_L, dtype=np.uint32)[None, :]
_P = ((_R + 6) & 7) >> 2
_CA = (_R + 2) & 3
_IDXC = ((_P << 16) | (_LN << 3) | (_CA + 2)) + _KS1
del _R, _LN, _P, _CA

_PAIRS = 8


def _sample_kernel(lg_ref, ic_ref, out_ref):
    i = pl.program_id(0)
    shape = (_C, _L)
    row = jax.lax.broadcasted_iota(jnp.uint32, shape, 0)

    lg = lg_ref[...]
    act = (row >= np.uint32(2)) & (row < np.uint32(6))
    m4 = jnp.where(act, lg, jnp.roll(lg, 4, axis=0))
    w4 = jnp.exp(-m4)
    ic = ic_ref[...]

    for u in range(_PAIRS):
        base = ((i * _PAIRS + u) * (2 * _L * _C)).astype(jnp.uint32)
        bits = _threefry_bits(ic + base)
        fbits = (bits >> np.uint32(9)) | np.uint32(0x3F800000)
        floats = (jax.lax.bitcast_convert_type(fbits, jnp.float32)
                  - np.float32(1.0))
        u01 = floats + _TINY
        s = jnp.log(u01) * w4

        m = s
        for k, sel in ((1, (row & np.uint32(1)) == 0),
                       (2, ((row + np.uint32(2)) & np.uint32(2)) == 0)):
            fwd = jnp.roll(m, -k, axis=0)
            bwd = jnp.roll(m, k, axis=0)
            m = jnp.maximum(m, jnp.where(sel, fwd, bwd))

        oh = jnp.where(s == m, np.float32(1.0), np.float32(0.0))
        out_ref[2 * u] = jnp.where(act, oh, np.float32(0.0))
        out_ref[2 * u + 1] = jnp.where(act, jnp.roll(oh, 4, axis=0),
                                       np.float32(0.0))


def kernel(logits):
    lg = jnp.transpose(logits[0])
    out = pl.pallas_call(
        _sample_kernel,
        grid=(_NS // (2 * _PAIRS),),
        in_specs=[pl.BlockSpec((_C, _L), lambda i: (0, 0)),
                  pl.BlockSpec((_C, _L), lambda i: (0, 0))],
        out_specs=pl.BlockSpec((2 * _PAIRS, _C, _L), lambda i: (i, 0, 0)),
        out_shape=jax.ShapeDtypeStruct((_NS, _C, _L), jnp.float32),
    )(lg, jnp.asarray(_IDXC))
    return jnp.transpose(out, (0, 2, 1)).reshape(_B, _NS, _L, _C)


# speedup vs baseline: 1.0255x; 1.0013x over previous
import jax
import jax.numpy as jnp
import numpy as np
from jax.experimental import pallas as pl
from jax.experimental.pallas import tpu as pltpu

_B, _L, _C, _NS = 1, 8192, 8, 64

_KS0 = np.uint32(0)
_KS1 = np.uint32(1234)
_KS2 = np.uint32(_KS0 ^ _KS1 ^ np.uint32(0x1BD11BDA))
_TINY = np.float32(np.finfo(np.float32).tiny)
_ROT_A = (13, 15, 26, 6)
_ROT_B = (17, 29, 16, 24)


def _rotl(x, r):
    return (x << np.uint32(r)) | (x >> np.uint32(32 - r))


def _threefry_rounds(x0, x1, rots):
    for r in rots:
        x0 = x0 + x1
        x1 = _rotl(x1, r)
        x1 = x0 ^ x1
    return x0, x1


def _threefry_bits(x1):
    x0 = jnp.full(x1.shape, _KS0, jnp.uint32)
    x0, x1 = _threefry_rounds(x0, x1, _ROT_A)
    x0, x1 = x0 + _KS1, x1 + (_KS2 + np.uint32(1))
    x0, x1 = _threefry_rounds(x0, x1, _ROT_B)
    x0, x1 = x0 + _KS2, x1 + (_KS0 + np.uint32(2))
    x0, x1 = _threefry_rounds(x0, x1, _ROT_A)
    x0, x1 = x0 + _KS0, x1 + (_KS1 + np.uint32(3))
    x0, x1 = _threefry_rounds(x0, x1, _ROT_B)
    x0, x1 = x0 + _KS1, x1 + (_KS2 + np.uint32(4))
    x0, x1 = _threefry_rounds(x0, x1, _ROT_A)
    x0, x1 = x0 + _KS2, x1 + (_KS0 + np.uint32(5))
    return x0 ^ x1


_R = np.arange(_C, dtype=np.uint32)[:, None]
_LN = np.arange(_L, dtype=np.uint32)[None, :]
_P = ((_R + 6) & 7) >> 2
_CA = (_R + 2) & 3
_IDXC = ((_P << 16) | (_LN << 3) | (_CA + 2)) + _KS1
del _R, _LN, _P, _CA

_PAIRS = 8


def _sample_kernel(lg_ref, ic_ref, out_ref):
    i = pl.program_id(0)
    shape = (_C, _L)
    row = jax.lax.broadcasted_iota(jnp.uint32, shape, 0)

    lg = lg_ref[...]
    act = (row >= np.uint32(2)) & (row < np.uint32(6))
    m4 = jnp.where(act, lg, jnp.roll(lg, 4, axis=0))
    w4 = jnp.exp(-m4)
    ic = ic_ref[...]

    for u in range(_PAIRS):
        base = ((i * _PAIRS + u) * (2 * _L * _C)).astype(jnp.uint32)
        bits = _threefry_bits(ic + base)
        fbits = (bits >> np.uint32(9)) | np.uint32(0x3F800000)
        floats = (jax.lax.bitcast_convert_type(fbits, jnp.float32)
                  - np.float32(1.0))
        u01 = floats + _TINY
        s = jnp.log(u01) * w4

        m = s
        for k, sel in ((1, (row & np.uint32(1)) == 0),
                       (2, ((row + np.uint32(2)) & np.uint32(2)) == 0)):
            fwd = jnp.roll(m, -k, axis=0)
            bwd = jnp.roll(m, k, axis=0)
            m = jnp.maximum(m, jnp.where(sel, fwd, bwd))

        oh = jnp.where(s == m, np.float32(1.0), np.float32(0.0))
        out_ref[2 * u] = jnp.where(act, oh, np.float32(0.0))
        out_ref[2 * u + 1] = jnp.where(act, jnp.roll(oh, 4, axis=0),
                                       np.float32(0.0))


def kernel(logits):
    lg = jnp.transpose(logits[0])
    out = pl.pallas_call(
        _sample_kernel,
        grid=(_NS // (2 * _PAIRS),),
        in_specs=[pl.BlockSpec((_C, _L), lambda i: (0, 0)),
                  pl.BlockSpec((_C, _L), lambda i: (0, 0))],
        out_specs=pl.BlockSpec((2 * _PAIRS, _C, _L), lambda i: (i, 0, 0)),
        out_shape=jax.ShapeDtypeStruct((_NS, _C, _L), jnp.float32),
        compiler_params=pltpu.CompilerParams(
            dimension_semantics=("parallel",)),
    )(lg, jnp.asarray(_IDXC))
    return jnp.transpose(out, (0, 2, 1)).reshape(_B, _NS, _L, _C)


# speedup vs baseline: 1.0329x; 1.0072x over previous
import jax
import jax.numpy as jnp
import numpy as np
from jax.experimental import pallas as pl
from jax.experimental.pallas import tpu as pltpu

_B, _L, _C, _NS = 1, 8192, 8, 64

_KS0 = np.uint32(0)
_KS1 = np.uint32(1234)
_KS2 = np.uint32(_KS0 ^ _KS1 ^ np.uint32(0x1BD11BDA))
_TINY = np.float32(np.finfo(np.float32).tiny)
_ROT_A = (13, 15, 26, 6)
_ROT_B = (17, 29, 16, 24)


def _rotl(x, r):
    return (x << np.uint32(r)) | (x >> np.uint32(32 - r))


def _threefry_rounds(x0, x1, rots):
    for r in rots:
        x0 = x0 + x1
        x1 = _rotl(x1, r)
        x1 = x0 ^ x1
    return x0, x1


def _threefry_bits(x1):
    x0 = jnp.full(x1.shape, _KS0, jnp.uint32)
    x0, x1 = _threefry_rounds(x0, x1, _ROT_A)
    x0, x1 = x0 + _KS1, x1 + (_KS2 + np.uint32(1))
    x0, x1 = _threefry_rounds(x0, x1, _ROT_B)
    x0, x1 = x0 + _KS2, x1 + (_KS0 + np.uint32(2))
    x0, x1 = _threefry_rounds(x0, x1, _ROT_A)
    x0, x1 = x0 + _KS0, x1 + (_KS1 + np.uint32(3))
    x0, x1 = _threefry_rounds(x0, x1, _ROT_B)
    x0, x1 = x0 + _KS1, x1 + (_KS2 + np.uint32(4))
    x0, x1 = _threefry_rounds(x0, x1, _ROT_A)
    x0, x1 = x0 + _KS2, x1 + (_KS0 + np.uint32(5))
    return x0 ^ x1


_R = np.arange(_C, dtype=np.uint32)[:, None]
_LN = np.arange(_L, dtype=np.uint32)[None, :]
_P = ((_R + 6) & 7) >> 2
_CA = (_R + 2) & 3
_IDXC = ((_P << 16) | (_LN << 3) | (_CA + 2)) + _KS1
del _R, _LN, _P, _CA

_PAIRS = 4


def _sample_kernel(lg_ref, ic_ref, out_ref):
    i = pl.program_id(0)
    shape = (_C, _L)
    row = jax.lax.broadcasted_iota(jnp.uint32, shape, 0)

    lg = lg_ref[...]
    act = (row >= np.uint32(2)) & (row < np.uint32(6))
    m4 = jnp.where(act, lg, jnp.roll(lg, 4, axis=0))
    w4 = jnp.exp(-m4)
    ic = ic_ref[...]

    for u in range(_PAIRS):
        base = ((i * _PAIRS + u) * (2 * _L * _C)).astype(jnp.uint32)
        bits = _threefry_bits(ic + base)
        fbits = (bits >> np.uint32(9)) | np.uint32(0x3F800000)
        floats = (jax.lax.bitcast_convert_type(fbits, jnp.float32)
                  - np.float32(1.0))
        u01 = floats + _TINY
        s = jnp.log(u01) * w4

        m = s
        for k, sel in ((1, (row & np.uint32(1)) == 0),
                       (2, ((row + np.uint32(2)) & np.uint32(2)) == 0)):
            fwd = jnp.roll(m, -k, axis=0)
            bwd = jnp.roll(m, k, axis=0)
            m = jnp.maximum(m, jnp.where(sel, fwd, bwd))

        oh = jnp.where(s == m, np.float32(1.0), np.float32(0.0))
        out_ref[2 * u] = jnp.where(act, oh, np.float32(0.0))
        out_ref[2 * u + 1] = jnp.where(act, jnp.roll(oh, 4, axis=0),
                                       np.float32(0.0))


def kernel(logits):
    lg = jnp.transpose(logits[0])
    out = pl.pallas_call(
        _sample_kernel,
        grid=(_NS // (2 * _PAIRS),),
        in_specs=[pl.BlockSpec((_C, _L), lambda i: (0, 0)),
                  pl.BlockSpec((_C, _L), lambda i: (0, 0))],
        out_specs=pl.BlockSpec((2 * _PAIRS, _C, _L), lambda i: (i, 0, 0)),
        out_shape=jax.ShapeDtypeStruct((_NS, _C, _L), jnp.float32),
        compiler_params=pltpu.CompilerParams(
            dimension_semantics=("parallel",)),
    )(lg, jnp.asarray(_IDXC))
    return jnp.transpose(out, (0, 2, 1)).reshape(_B, _NS, _L, _C)
